# baseline (speedup 1.0000x reference)
import functools

import jax
import jax.numpy as jnp
from jax import lax
from jax.experimental import pallas as pl
from jax.experimental.pallas import tpu as pltpu
from jax.experimental.pallas import tpu_sc as plsc

N = 10000
D = 128
HD = 64
E = 320000

NC = 2
NS = 16
NP = 10240
ET = 10752
EP = NC * NS * ET
NBLK = 2 * ET // 128 // 8
NBLK_C0 = 11
ROWS_PER_TILE = NP // NS


def _mm_body(x_ref, w_ref, av_ref, hlo_ref, hhi_ref, als_ref):
    h = jnp.dot(x_ref[...], w_ref[...], preferred_element_type=jnp.float32)
    hlo_ref[...] = h[:, :HD]
    hhi_ref[...] = h[:, HD:]
    als_ref[...] = jnp.dot(h, av_ref[...], preferred_element_type=jnp.float32)


def _tc_mm(x, w, av, blk=640):
    n = x.shape[0]
    return pl.pallas_call(
        _mm_body,
        grid=(n // blk,),
        in_specs=[
            pl.BlockSpec((blk, D), lambda i: (i, 0)),
            pl.BlockSpec((D, D), lambda i: (0, 0)),
            pl.BlockSpec((D, 2), lambda i: (0, 0)),
        ],
        out_specs=[
            pl.BlockSpec((blk, HD), lambda i: (i, 0)),
            pl.BlockSpec((blk, HD), lambda i: (i, 0)),
            pl.BlockSpec((blk, 2), lambda i: (i, 0)),
        ],
        out_shape=[
            jax.ShapeDtypeStruct((n, HD), jnp.float32),
            jax.ShapeDtypeStruct((n, HD), jnp.float32),
            jax.ShapeDtypeStruct((n, 2), jnp.float32),
        ],
    )(x, w, av)


def _combine(p00, p01, p10, p11, s, b):
    den = s[:, 0:1] + s[:, 1:2] + 1e-16
    vl = (p00 + p10) / den + b[:, :HD]
    vr = (p01 + p11) / den + b[:, HD:]
    return jnp.concatenate([vl, vr], axis=1)


def _mid_body(p00_ref, p01_ref, p10_ref, p11_ref, s_ref, b_ref, w_ref,
              av_ref, hlo_ref, hhi_ref, als_ref):
    v = _combine(p00_ref[...], p01_ref[...], p10_ref[...], p11_ref[...],
                 s_ref[...], b_ref[...])
    h1e = jnp.where(v > 0, v, jnp.exp(jnp.minimum(v, 0.0)) - 1.0)
    h = jnp.dot(h1e, w_ref[...], preferred_element_type=jnp.float32)
    hlo_ref[...] = h[:, :HD]
    hhi_ref[...] = h[:, HD:]
    als_ref[...] = jnp.dot(h, av_ref[...], preferred_element_type=jnp.float32)


def _tc_mid(p00, p01, p10, p11, s2, b, w, av, blk=640):
    n = p00.shape[0]
    pspec = pl.BlockSpec((blk, HD), lambda i: (i, 0))
    return pl.pallas_call(
        _mid_body,
        grid=(n // blk,),
        in_specs=[
            pspec, pspec, pspec, pspec,
            pl.BlockSpec((blk, 2), lambda i: (i, 0)),
            pl.BlockSpec((1, D), lambda i: (0, 0)),
            pl.BlockSpec((D, D), lambda i: (0, 0)),
            pl.BlockSpec((D, 2), lambda i: (0, 0)),
        ],
        out_specs=[
            pl.BlockSpec((blk, HD), lambda i: (i, 0)),
            pl.BlockSpec((blk, HD), lambda i: (i, 0)),
            pl.BlockSpec((blk, 2), lambda i: (i, 0)),
        ],
        out_shape=[
            jax.ShapeDtypeStruct((n, HD), jnp.float32),
            jax.ShapeDtypeStruct((n, HD), jnp.float32),
            jax.ShapeDtypeStruct((n, 2), jnp.float32),
        ],
    )(p00, p01, p10, p11, s2, b, w, av)


def _fin_body(p00_ref, p01_ref, p10_ref, p11_ref, s_ref, b_ref, o_ref):
    o_ref[...] = _combine(p00_ref[...], p01_ref[...], p10_ref[...],
                          p11_ref[...], s_ref[...], b_ref[...])


def _tc_fin(p00, p01, p10, p11, s2, b, blk=640):
    n = p00.shape[0]
    pspec = pl.BlockSpec((blk, HD), lambda i: (i, 0))
    return pl.pallas_call(
        _fin_body,
        grid=(n // blk,),
        in_specs=[
            pspec, pspec, pspec, pspec,
            pl.BlockSpec((blk, 2), lambda i: (i, 0)),
            pl.BlockSpec((1, D), lambda i: (0, 0)),
        ],
        out_specs=pl.BlockSpec((blk, D), lambda i: (i, 0)),
        out_shape=jax.ShapeDtypeStruct((n, D), jnp.float32),
    )(p00, p01, p10, p11, s2, b)


def _sc_body(hlo_hbm, hhi_hbm, als_hbm, ald_hbm, src_hbm, dst_hbm,
             pout_hbm, sden_hbm,
             als_v, ald_v, src_b, dst_b, ex_t, bufa, bufb, bufc,
             sem_s, sem_a, sem_b, sem_c, s_sp, h_sp, out_sp):
    c = lax.axis_index("c")
    sid = lax.axis_index("s")
    off = sid * ROWS_PER_TILE

    pltpu.sync_copy(als_hbm, als_v)
    pltpu.sync_copy(ald_hbm, ald_v)

    z16 = jnp.zeros((16,), jnp.float32)

    def _zero_bufa():
        def _zb(r, carry):
            for u in range(4):
                bufa[r, pl.ds(u * 16, 16)] = z16
            return carry
        lax.fori_loop(0, 128, _zb, 0)

    def _zero_out_stripe():
        for i in range(ROWS_PER_TILE // 128):
            pltpu.sync_copy(bufa, out_sp.at[pl.ds(off + i * 128, 128)])

    _zero_bufa()
    _zero_out_stripe()
    pltpu.sync_copy(hlo_hbm.at[pl.ds(off, ROWS_PER_TILE)],
                    h_sp.at[pl.ds(off, ROWS_PER_TILE)])

    @pl.when(sid == 0)
    def _():
        def _zero_s(i, carry):
            ex_t[pl.ds(i * 16, 16)] = z16
            return carry
        lax.fori_loop(0, 1024 // 16, _zero_s, 0)
        for i in range(NP // ROWS_PER_TILE):
            pltpu.sync_copy(ex_t.at[pl.ds(0, ROWS_PER_TILE)],
                            s_sp.at[pl.ds(i * ROWS_PER_TILE, ROWS_PER_TILE)])

    plsc.subcore_barrier()

    nb = jnp.where(c == 0, NBLK_C0, NBLK - NBLK_C0)

    def _logits(jj):
        eoff = jj * 128
        for u in range(8):
            si = src_b[jj, pl.ds(u * 16, 16)]
            di = dst_b[jj, pl.ds(u * 16, 16)]
            e = (plsc.load_gather(als_v, [si])
                 + plsc.load_gather(ald_v, [di]))
            e = jnp.where(e >= 0, e, 0.2 * e)
            ex_t[pl.ds(eoff + u * 16, 16)] = jnp.exp(e)

    def _scale(buf, jj):
        eoff = jj * 128

        def _scale_g(g, carry2):
            a16 = ex_t[pl.ds(eoff + g * 16, 16)]
            for k in range(16):
                r = g * 16 + k
                av = jnp.full((16,), a16[k])
                for u in range(4):
                    buf[r, pl.ds(u * 16, 16)] = buf[r, pl.ds(u * 16, 16)] * av
            return carry2
        lax.fori_loop(0, 8, _scale_g, 0)

    def _wait_g(jj, buf, sem):
        pltpu.make_async_copy(h_sp.at[src_b.at[jj]], buf, sem).wait()

    def _wait_w(jj, buf, sem):
        pltpu.make_async_copy(buf, out_sp.at[dst_b.at[jj]], sem).wait()

    def _run_pass(first):
        def _block(bi, carry):
            goff = (sid * NBLK + c * NBLK_C0 + bi) * 8
            pltpu.sync_copy(src_hbm.at[pl.ds(goff, 8)], src_b)
            pltpu.sync_copy(dst_hbm.at[pl.ds(goff, 8)], dst_b)
            for jj in range(8):
                _logits(jj)
                if first:
                    pltpu.async_copy(ex_t.at[pl.ds(jj * 128, 128)],
                                     s_sp.at[dst_b.at[jj]], add=True,
                                     sem=sem_s)
            bufs = [bufa, bufb, bufc]
            sems = [sem_a, sem_b, sem_c]
            pltpu.async_copy(h_sp.at[src_b.at[0]], bufs[0], sems[0])
            for j in range(8):
                if j + 1 <= 7:
                    if j >= 2:
                        _wait_w(j - 2, bufs[(j + 1) % 3], sems[(j + 1) % 3])
                    pltpu.async_copy(h_sp.at[src_b.at[j + 1]],
                                     bufs[(j + 1) % 3], sems[(j + 1) % 3])
                _wait_g(j, bufs[j % 3], sems[j % 3])
                _scale(bufs[j % 3], j)
                pltpu.async_copy(bufs[j % 3], out_sp.at[dst_b.at[j]],
                                 add=True, sem=sems[j % 3])
            for j in range(5, 8):
                _wait_w(j, bufs[j % 3], sems[j % 3])
            if first:
                for jj in range(8):
                    pltpu.make_async_copy(
                        ex_t.at[pl.ds(jj * 128, 128)],
                        s_sp.at[dst_b.at[jj]], sem_s).wait()
            return carry

        lax.fori_loop(0, nb, _block, 0)

    _run_pass(True)
    plsc.subcore_barrier()

    pltpu.sync_copy(out_sp.at[pl.ds(off, ROWS_PER_TILE)],
                    pout_hbm.at[c, 0, pl.ds(off, ROWS_PER_TILE)])
    _zero_bufa()
    _zero_out_stripe()
    pltpu.sync_copy(hhi_hbm.at[pl.ds(off, ROWS_PER_TILE)],
                    h_sp.at[pl.ds(off, ROWS_PER_TILE)])
    plsc.subcore_barrier()

    _run_pass(False)
    plsc.subcore_barrier()

    pltpu.sync_copy(out_sp.at[pl.ds(off, ROWS_PER_TILE)],
                    pout_hbm.at[c, 1, pl.ds(off, ROWS_PER_TILE)])
    pltpu.sync_copy(s_sp.at[pl.ds(off, ROWS_PER_TILE)],
                    sden_hbm.at[c, pl.ds(off, ROWS_PER_TILE)])


@functools.partial(jax.jit, static_argnames=())
def _sc_layer(hlo, hhi, als_pad, ald_pad, src2, dst2):
    mesh = plsc.VectorSubcoreMesh(core_axis_name="c", subcore_axis_name="s")
    kern = pl.kernel(
        _sc_body,
        out_type=[
            jax.ShapeDtypeStruct((NC, 2, NP, HD), jnp.float32),
            jax.ShapeDtypeStruct((NC, NP), jnp.float32),
        ],
        mesh=mesh,
        compiler_params=pltpu.CompilerParams(needs_layout_passes=False,
                                             use_tc_tiling_on_sc=False),
        scratch_types=[
            pltpu.VMEM((NP,), jnp.float32),
            pltpu.VMEM((NP,), jnp.float32),
            pltpu.VMEM((8, 128), jnp.int32),
            pltpu.VMEM((8, 128), jnp.int32),
            pltpu.VMEM((8 * 128,), jnp.float32),
            pltpu.VMEM((128, HD), jnp.float32),
            pltpu.VMEM((128, HD), jnp.float32),
            pltpu.VMEM((128, HD), jnp.float32),
            pltpu.SemaphoreType.DMA,
            pltpu.SemaphoreType.DMA,
            pltpu.SemaphoreType.DMA,
            pltpu.SemaphoreType.DMA,
            pltpu.VMEM_SHARED((NP,), jnp.float32),
            pltpu.VMEM_SHARED((NP, HD), jnp.float32),
            pltpu.VMEM_SHARED((NP, HD), jnp.float32),
        ],
    )
    return kern(hlo, hhi, als_pad, ald_pad, src2, dst2)


def kernel(x, edge_index, W1, a1_src, a1_dst, b1, W2, a2_src, a2_dst, b2):
    sl = jnp.arange(N, dtype=edge_index.dtype)
    npad = EP - E - N
    src = jnp.concatenate([edge_index[0], sl,
                           jnp.zeros((npad,), edge_index.dtype)])
    dst = jnp.concatenate([edge_index[1], sl,
                           jnp.full((npad,), N, edge_index.dtype)])
    src2 = src.reshape(EP // 128, 128)
    dst2 = dst.reshape(EP // 128, 128)

    av1 = jnp.stack([a1_src[0], a1_dst[0]], axis=-1)
    av2 = jnp.stack([a2_src[0], a2_dst[0]], axis=-1)
    x_pad = jnp.pad(x, ((0, NP - N), (0, 0)))

    h1lo, h1hi, als1 = _tc_mm(x_pad, W1, av1)
    p1, s1 = _sc_layer(h1lo, h1hi, als1[:, 0], als1[:, 1], src2, dst2)
    s1t = jnp.stack([s1[0], s1[1]], axis=-1)

    h2lo, h2hi, als2 = _tc_mid(p1[0, 0], p1[0, 1], p1[1, 0], p1[1, 1],
                               s1t, b1.reshape(1, D), W2, av2)
    p2, s2 = _sc_layer(h2lo, h2hi, als2[:, 0], als2[:, 1], src2, dst2)
    s2t = jnp.stack([s2[0], s2[1]], axis=-1)

    out = _tc_fin(p2[0, 0], p2[0, 1], p2[1, 0], p2[1, 1],
                  s2t, b2.reshape(1, D))
    return out[:N]


# speedup vs baseline: 1.0023x; 1.0023x over previous
import functools

import jax
import jax.numpy as jnp
from jax import lax
from jax.experimental import pallas as pl
from jax.experimental.pallas import tpu as pltpu
from jax.experimental.pallas import tpu_sc as plsc

N = 10000
D = 128
HD = 64
E = 320000

NC = 2
NS = 16
NP = 10240
ET = 10752
EP = NC * NS * ET
NBLK = 2 * ET // 64 // 16
NBLK_C0 = 11
ROWS_PER_TILE = NP // NS


def _mm_body(x_ref, w_ref, av_ref, hlo_ref, hhi_ref, als_ref):
    h = jnp.dot(x_ref[...], w_ref[...], preferred_element_type=jnp.float32)
    hlo_ref[...] = h[:, :HD]
    hhi_ref[...] = h[:, HD:]
    als_ref[...] = jnp.dot(h, av_ref[...], preferred_element_type=jnp.float32)


def _tc_mm(x, w, av, blk=640):
    n = x.shape[0]
    return pl.pallas_call(
        _mm_body,
        grid=(n // blk,),
        in_specs=[
            pl.BlockSpec((blk, D), lambda i: (i, 0)),
            pl.BlockSpec((D, D), lambda i: (0, 0)),
            pl.BlockSpec((D, 2), lambda i: (0, 0)),
        ],
        out_specs=[
            pl.BlockSpec((blk, HD), lambda i: (i, 0)),
            pl.BlockSpec((blk, HD), lambda i: (i, 0)),
            pl.BlockSpec((blk, 2), lambda i: (i, 0)),
        ],
        out_shape=[
            jax.ShapeDtypeStruct((n, HD), jnp.float32),
            jax.ShapeDtypeStruct((n, HD), jnp.float32),
            jax.ShapeDtypeStruct((n, 2), jnp.float32),
        ],
    )(x, w, av)


def _combine(p00, p01, p10, p11, s, b):
    den = s[:, 0:1] + s[:, 1:2] + 1e-16
    vl = (p00 + p10) / den + b[:, :HD]
    vr = (p01 + p11) / den + b[:, HD:]
    return jnp.concatenate([vl, vr], axis=1)


def _mid_body(p00_ref, p01_ref, p10_ref, p11_ref, s_ref, b_ref, w_ref,
              av_ref, hlo_ref, hhi_ref, als_ref):
    v = _combine(p00_ref[...], p01_ref[...], p10_ref[...], p11_ref[...],
                 s_ref[...], b_ref[...])
    h1e = jnp.where(v > 0, v, jnp.exp(jnp.minimum(v, 0.0)) - 1.0)
    h = jnp.dot(h1e, w_ref[...], preferred_element_type=jnp.float32)
    hlo_ref[...] = h[:, :HD]
    hhi_ref[...] = h[:, HD:]
    als_ref[...] = jnp.dot(h, av_ref[...], preferred_element_type=jnp.float32)


def _tc_mid(p00, p01, p10, p11, s2, b, w, av, blk=640):
    n = p00.shape[0]
    pspec = pl.BlockSpec((blk, HD), lambda i: (i, 0))
    return pl.pallas_call(
        _mid_body,
        grid=(n // blk,),
        in_specs=[
            pspec, pspec, pspec, pspec,
            pl.BlockSpec((blk, 2), lambda i: (i, 0)),
            pl.BlockSpec((1, D), lambda i: (0, 0)),
            pl.BlockSpec((D, D), lambda i: (0, 0)),
            pl.BlockSpec((D, 2), lambda i: (0, 0)),
        ],
        out_specs=[
            pl.BlockSpec((blk, HD), lambda i: (i, 0)),
            pl.BlockSpec((blk, HD), lambda i: (i, 0)),
            pl.BlockSpec((blk, 2), lambda i: (i, 0)),
        ],
        out_shape=[
            jax.ShapeDtypeStruct((n, HD), jnp.float32),
            jax.ShapeDtypeStruct((n, HD), jnp.float32),
            jax.ShapeDtypeStruct((n, 2), jnp.float32),
        ],
    )(p00, p01, p10, p11, s2, b, w, av)


def _fin_body(p00_ref, p01_ref, p10_ref, p11_ref, s_ref, b_ref, o_ref):
    o_ref[...] = _combine(p00_ref[...], p01_ref[...], p10_ref[...],
                          p11_ref[...], s_ref[...], b_ref[...])


def _tc_fin(p00, p01, p10, p11, s2, b, blk=640):
    n = p00.shape[0]
    pspec = pl.BlockSpec((blk, HD), lambda i: (i, 0))
    return pl.pallas_call(
        _fin_body,
        grid=(n // blk,),
        in_specs=[
            pspec, pspec, pspec, pspec,
            pl.BlockSpec((blk, 2), lambda i: (i, 0)),
            pl.BlockSpec((1, D), lambda i: (0, 0)),
        ],
        out_specs=pl.BlockSpec((blk, D), lambda i: (i, 0)),
        out_shape=jax.ShapeDtypeStruct((n, D), jnp.float32),
    )(p00, p01, p10, p11, s2, b)


def _sc_body(hlo_hbm, hhi_hbm, als_hbm, ald_hbm, src_hbm, dst_hbm,
             pout_hbm, sden_hbm,
             als_v, ald_v, src_b, dst_b, ex_t, zs, bufa, bufb, bufc, bufd,
             sem_s, sem_a, sem_b, sem_c, sem_d, s_sp, h_sp, out_sp):
    c = lax.axis_index("c")
    sid = lax.axis_index("s")
    off = sid * ROWS_PER_TILE

    pltpu.sync_copy(als_hbm, als_v)
    pltpu.sync_copy(ald_hbm, ald_v)

    z16 = jnp.zeros((16,), jnp.float32)

    def _zero_bufa():
        def _zb(r, carry):
            for u in range(4):
                bufa[r, pl.ds(u * 16, 16)] = z16
            return carry
        lax.fori_loop(0, 64, _zb, 0)

    def _zero_out_stripe():
        for i in range(ROWS_PER_TILE // 64):
            pltpu.sync_copy(bufa, out_sp.at[pl.ds(off + i * 64, 64)])

    _zero_bufa()
    _zero_out_stripe()
    pltpu.sync_copy(hlo_hbm.at[pl.ds(off, ROWS_PER_TILE)],
                    h_sp.at[pl.ds(off, ROWS_PER_TILE)])

    @pl.when(sid == 0)
    def _():
        def _zero_s(i, carry):
            zs[pl.ds(i * 16, 16)] = z16
            return carry
        lax.fori_loop(0, ROWS_PER_TILE // 16, _zero_s, 0)
        for i in range(NP // ROWS_PER_TILE):
            pltpu.sync_copy(zs, s_sp.at[pl.ds(i * ROWS_PER_TILE, ROWS_PER_TILE)])

    plsc.subcore_barrier()

    nb = jnp.where(c == 0, NBLK_C0, NBLK - NBLK_C0)

    def _logits(jj):
        eoff = jj * 64
        for u in range(4):
            si = src_b[jj, pl.ds(u * 16, 16)]
            di = dst_b[jj, pl.ds(u * 16, 16)]
            e = (plsc.load_gather(als_v, [si])
                 + plsc.load_gather(ald_v, [di]))
            e = jnp.where(e >= 0, e, 0.2 * e)
            ex_t[pl.ds(eoff + u * 16, 16)] = jnp.exp(e)

    def _scale(buf, jj):
        eoff = jj * 64

        def _scale_g(g, carry2):
            a16 = ex_t[pl.ds(eoff + g * 16, 16)]
            for k in range(16):
                r = g * 16 + k
                av = jnp.full((16,), a16[k])
                for u in range(4):
                    buf[r, pl.ds(u * 16, 16)] = buf[r, pl.ds(u * 16, 16)] * av
            return carry2
        lax.fori_loop(0, 4, _scale_g, 0)

    def _wait_g(jj, buf, sem):
        pltpu.make_async_copy(h_sp.at[src_b.at[jj]], buf, sem).wait()

    def _wait_w(jj, buf, sem):
        pltpu.make_async_copy(buf, out_sp.at[dst_b.at[jj]], sem).wait()

    def _run_pass(first):
        def _block(bi, carry):
            goff = (sid * NBLK + c * NBLK_C0 + bi) * 16
            pltpu.sync_copy(src_hbm.at[pl.ds(goff, 16)], src_b)
            pltpu.sync_copy(dst_hbm.at[pl.ds(goff, 16)], dst_b)
            for jj in range(16):
                _logits(jj)
                if first:
                    pltpu.async_copy(ex_t.at[pl.ds(jj * 64, 64)],
                                     s_sp.at[dst_b.at[jj]], add=True,
                                     sem=sem_s)
            bufs = [bufa, bufb, bufc, bufd]
            sems = [sem_a, sem_b, sem_c, sem_d]
            pltpu.async_copy(h_sp.at[src_b.at[0]], bufs[0], sems[0])
            pltpu.async_copy(h_sp.at[src_b.at[1]], bufs[1], sems[1])
            for j in range(16):
                _wait_g(j, bufs[j % 4], sems[j % 4])
                _scale(bufs[j % 4], j)
                if j + 2 <= 15:
                    if j >= 2:
                        _wait_w(j - 2, bufs[(j + 2) % 4], sems[(j + 2) % 4])
                    pltpu.async_copy(h_sp.at[src_b.at[j + 2]],
                                     bufs[(j + 2) % 4], sems[(j + 2) % 4])
                pltpu.async_copy(bufs[j % 4], out_sp.at[dst_b.at[j]],
                                 add=True, sem=sems[j % 4])
            for j in range(12, 16):
                _wait_w(j, bufs[j % 4], sems[j % 4])
            if first:
                for jj in range(16):
                    pltpu.make_async_copy(
                        ex_t.at[pl.ds(jj * 64, 64)],
                        s_sp.at[dst_b.at[jj]], sem_s).wait()
            return carry

        lax.fori_loop(0, nb, _block, 0)

    _run_pass(True)
    plsc.subcore_barrier()

    pltpu.sync_copy(out_sp.at[pl.ds(off, ROWS_PER_TILE)],
                    pout_hbm.at[c, 0, pl.ds(off, ROWS_PER_TILE)])
    _zero_bufa()
    _zero_out_stripe()
    pltpu.sync_copy(hhi_hbm.at[pl.ds(off, ROWS_PER_TILE)],
                    h_sp.at[pl.ds(off, ROWS_PER_TILE)])
    plsc.subcore_barrier()

    _run_pass(False)
    plsc.subcore_barrier()

    pltpu.sync_copy(out_sp.at[pl.ds(off, ROWS_PER_TILE)],
                    pout_hbm.at[c, 1, pl.ds(off, ROWS_PER_TILE)])
    pltpu.sync_copy(s_sp.at[pl.ds(off, ROWS_PER_TILE)],
                    sden_hbm.at[c, pl.ds(off, ROWS_PER_TILE)])


@functools.partial(jax.jit, static_argnames=())
def _sc_layer(hlo, hhi, als_pad, ald_pad, src2, dst2):
    mesh = plsc.VectorSubcoreMesh(core_axis_name="c", subcore_axis_name="s")
    kern = pl.kernel(
        _sc_body,
        out_type=[
            jax.ShapeDtypeStruct((NC, 2, NP, HD), jnp.float32),
            jax.ShapeDtypeStruct((NC, NP), jnp.float32),
        ],
        mesh=mesh,
        compiler_params=pltpu.CompilerParams(needs_layout_passes=False,
                                             use_tc_tiling_on_sc=False),
        scratch_types=[
            pltpu.VMEM((NP,), jnp.float32),
            pltpu.VMEM((NP,), jnp.float32),
            pltpu.VMEM((16, 64), jnp.int32),
            pltpu.VMEM((16, 64), jnp.int32),
            pltpu.VMEM((16 * 64,), jnp.float32),
            pltpu.VMEM((ROWS_PER_TILE,), jnp.float32),
            pltpu.VMEM((64, HD), jnp.float32),
            pltpu.VMEM((64, HD), jnp.float32),
            pltpu.VMEM((64, HD), jnp.float32),
            pltpu.VMEM((64, HD), jnp.float32),
            pltpu.SemaphoreType.DMA,
            pltpu.SemaphoreType.DMA,
            pltpu.SemaphoreType.DMA,
            pltpu.SemaphoreType.DMA,
            pltpu.SemaphoreType.DMA,
            pltpu.VMEM_SHARED((NP,), jnp.float32),
            pltpu.VMEM_SHARED((NP, HD), jnp.float32),
            pltpu.VMEM_SHARED((NP, HD), jnp.float32),
        ],
    )
    return kern(hlo, hhi, als_pad, ald_pad, src2, dst2)


def kernel(x, edge_index, W1, a1_src, a1_dst, b1, W2, a2_src, a2_dst, b2):
    sl = jnp.arange(N, dtype=edge_index.dtype)
    npad = EP - E - N
    src = jnp.concatenate([edge_index[0], sl,
                           jnp.zeros((npad,), edge_index.dtype)])
    dst = jnp.concatenate([edge_index[1], sl,
                           jnp.full((npad,), N, edge_index.dtype)])
    src2 = src.reshape(EP // 64, 64)
    dst2 = dst.reshape(EP // 64, 64)

    av1 = jnp.stack([a1_src[0], a1_dst[0]], axis=-1)
    av2 = jnp.stack([a2_src[0], a2_dst[0]], axis=-1)
    x_pad = jnp.pad(x, ((0, NP - N), (0, 0)))

    h1lo, h1hi, als1 = _tc_mm(x_pad, W1, av1)
    p1, s1 = _sc_layer(h1lo, h1hi, als1[:, 0], als1[:, 1], src2, dst2)
    s1t = jnp.stack([s1[0], s1[1]], axis=-1)

    h2lo, h2hi, als2 = _tc_mid(p1[0, 0], p1[0, 1], p1[1, 0], p1[1, 1],
                               s1t, b1.reshape(1, D), W2, av2)
    p2, s2 = _sc_layer(h2lo, h2hi, als2[:, 0], als2[:, 1], src2, dst2)
    s2t = jnp.stack([s2[0], s2[1]], axis=-1)

    out = _tc_fin(p2[0, 0], p2[0, 1], p2[1, 0], p2[1, 1],
                  s2t, b2.reshape(1, D))
    return out[:N]
